# baseline (speedup 1.0000x reference)
import functools

import jax
import jax.numpy as jnp
from jax import lax
from jax.experimental import pallas as pl
from jax.experimental.pallas import tpu as pltpu
from jax.experimental.pallas import tpu_sc as plsc

D = 768
MAXPOS = 1025
B = 64
L = 1024
N = B * L
EPS = 1e-12

NC, NS, LANES = 2, 16, 16
NW = NC * NS
K = 16
NCHUNK = L // K
NV = D // LANES
INV_D = 1.0 / D
ROWS_SH = 896


def _comb_body(pos_ref, type_ref, main_ref, pad_ref):
    main_ref[...] = pos_ref[pl.ds(1, L), :] + type_ref[1:2, :]
    pad_ref[...] = pos_ref[0:1, :] + type_ref[0:1, :]


_comb_call = pl.pallas_call(
    _comb_body,
    out_shape=(
        jax.ShapeDtypeStruct((L, D), jnp.float32),
        jax.ShapeDtypeStruct((1, D), jnp.float32),
    ),
)


def _rsqrt16(a):
    yi = plsc.bitcast(a, jnp.int32)
    magic = jnp.full((LANES,), 0x5F3759DF, dtype=jnp.int32)
    y = plsc.bitcast(magic - lax.shift_right_logical(yi, 1), jnp.float32)
    half = a * 0.5
    for _ in range(3):
        y = y * (1.5 - half * y * y)
    return y


def _splat_i32(x):
    return jnp.full((LANES,), x, dtype=jnp.int32)


_mesh = plsc.VectorSubcoreMesh(core_axis_name="c", subcore_axis_name="s")


@functools.partial(
    pl.kernel,
    mesh=_mesh,
    compiler_params=pltpu.CompilerParams(needs_layout_passes=False),
    out_type=jax.ShapeDtypeStruct((N, D), jnp.float32),
    scratch_types=[
        pltpu.VMEM((2, K), jnp.int32),
        pltpu.VMEM((2, K), jnp.int32),
        pltpu.VMEM((2, K, D), jnp.float32),
        pltpu.VMEM((2, K, D), jnp.float32),
        pltpu.VMEM((2, K, D), jnp.float32),
        pltpu.VMEM((1, D), jnp.float32),
        pltpu.VMEM((2, K, LANES), jnp.float32),
        pltpu.VMEM((2, K, LANES), jnp.float32),
        pltpu.VMEM((K,), jnp.float32),
        pltpu.VMEM((2, K), jnp.float32),
        pltpu.VMEM((2, K), jnp.float32),
        pltpu.VMEM((D,), jnp.float32),
        pltpu.VMEM((D,), jnp.float32),
        pltpu.VMEM_SHARED((ROWS_SH, D), jnp.float32),
        pltpu.SemaphoreType.DMA,
        pltpu.SemaphoreType.DMA,
        pltpu.SemaphoreType.DMA,
        pltpu.SemaphoreType.DMA,
        pltpu.SemaphoreType.DMA,
        pltpu.SemaphoreType.DMA,
        pltpu.SemaphoreType.DMA,
        pltpu.SemaphoreType.DMA,
        pltpu.SemaphoreType.DMA,
        pltpu.SemaphoreType.DMA,
    ],
)
def _sc_embed(ids_hbm, tok_hbm, comb_hbm, pad_hbm, gamma_hbm, beta_hbm, out_hbm,
              idsA_v, idsB_v, tokA_v, tokB_v, comb_v, row0_v, sb_v, ssb_v,
              mk_v, mean_v, inv_v, gam_v, bet_v, comb_sh,
              sa0, sa1, sb0, sb1, sc0, sc1, oa0, oa1, ob0, ob1):
    wid = lax.axis_index("s") * NC + lax.axis_index("c")
    baseA = wid * L
    baseB = (wid + NW) * L
    sem_tokA = (sa0, sa1)
    sem_tokB = (sb0, sb1)
    sem_comb = (sc0, sc1)
    sem_outA = (oa0, oa1)
    sem_outB = (ob0, ob1)
    pltpu.sync_copy(gamma_hbm, gam_v)
    pltpu.sync_copy(beta_hbm, bet_v)
    pltpu.sync_copy(pad_hbm, row0_v)

    @pl.when(lax.axis_index("s") == 0)
    def _load_comb():
        pltpu.sync_copy(comb_hbm.at[pl.ds(0, ROWS_SH)], comb_sh)

    plsc.subcore_barrier()

    zero = jnp.zeros((LANES,), jnp.float32)

    def fire_chunk(c, p):
        pb = c * K
        pltpu.sync_copy(ids_hbm.at[pl.ds(baseA + pb, K)], idsA_v.at[p])
        pltpu.sync_copy(ids_hbm.at[pl.ds(baseB + pb, K)], idsB_v.at[p])
        pltpu.async_copy(tok_hbm.at[idsA_v.at[p]], tokA_v.at[p], sem_tokA[p])
        pltpu.async_copy(tok_hbm.at[idsB_v.at[p]], tokB_v.at[p], sem_tokB[p])

        @pl.when(pb + K <= ROWS_SH)
        def _from_spmem():
            pltpu.async_copy(comb_sh.at[pl.ds(pb, K)], comb_v.at[p], sem_comb[p])

        @pl.when(pb + K > ROWS_SH)
        def _from_hbm():
            pltpu.async_copy(comb_hbm.at[pl.ds(pb, K)], comb_v.at[p], sem_comb[p])

    def wait_gathers(p):
        pltpu.make_async_copy(tok_hbm.at[pl.ds(0, K)], tokA_v.at[p], sem_tokA[p]).wait()
        pltpu.make_async_copy(tok_hbm.at[pl.ds(0, K)], tokB_v.at[p], sem_tokB[p]).wait()
        pltpu.make_async_copy(tok_hbm.at[pl.ds(0, K)], comb_v.at[p], sem_comb[p]).wait()

    def wait_outs(p):
        pltpu.make_async_copy(tok_hbm.at[pl.ds(0, K)], tokA_v.at[p], sem_outA[p]).wait()
        pltpu.make_async_copy(tok_hbm.at[pl.ds(0, K)], tokB_v.at[p], sem_outB[p]).wait()

    def compute_chunk(p):
        cv = comb_v.at[p]

        for ids_ref, tv in ((idsA_v.at[p], tokA_v.at[p]),
                            (idsB_v.at[p], tokB_v.at[p])):
            idv = ids_ref[...]
            is_pad = idv == 0
            mk_v[...] = jnp.where(is_pad, 1.0, 0.0)
            npad = plsc.all_reduce_population_count(is_pad)[0]

            @pl.when(npad != 0)
            def _fix_pads(_tv=tv):
                def fix(t, carry):
                    mt = plsc.load_gather(mk_v, [_splat_i32(t)]) != 0.0
                    for v in range(NV):
                        sl = pl.ds(v * LANES, LANES)
                        cur = _tv[t, sl]
                        fixed = cur + row0_v[0, sl] - cv[t, sl]
                        _tv[t, sl] = jnp.where(mt, fixed, cur)
                    return carry

                lax.fori_loop(0, K, fix, 0)

        tvA = tokA_v.at[p]
        tvB = tokB_v.at[p]

        def pass1(t, carry):
            sA = zero
            ssA = zero
            sB = zero
            ssB = zero
            for v in range(NV):
                sl = pl.ds(v * LANES, LANES)
                cb = cv[t, sl]
                xA = tvA[t, sl] + cb
                tvA[t, sl] = xA
                sA = sA + xA
                ssA = ssA + xA * xA
                xB = tvB[t, sl] + cb
                tvB[t, sl] = xB
                sB = sB + xB
                ssB = ssB + xB * xB
            sb_v[0, t, :] = sA
            ssb_v[0, t, :] = ssA
            sb_v[1, t, :] = sB
            ssb_v[1, t, :] = ssB
            return carry

        lax.fori_loop(0, K, pass1, 0)

        rows = lax.iota(jnp.int32, LANES)
        for r in (0, 1):
            s_tot = zero
            ss_tot = zero
            for j in range(LANES):
                col = _splat_i32(j)
                s_tot = s_tot + plsc.load_gather(sb_v.at[r], [rows, col])
                ss_tot = ss_tot + plsc.load_gather(ssb_v.at[r], [rows, col])
            mean = s_tot * INV_D
            var = ss_tot * INV_D - mean * mean
            mean_v[r, :] = mean
            inv_v[r, :] = _rsqrt16(var + EPS)

        NBLK = 4
        VB = NV // NBLK
        for blk in range(NBLK):
            gs = [gam_v[pl.ds((blk * VB + v) * LANES, LANES)] for v in range(VB)]
            bs = [bet_v[pl.ds((blk * VB + v) * LANES, LANES)] for v in range(VB)]
            for r, tv in ((0, tvA), (1, tvB)):

                def pass2(t, carry, _gs=gs, _bs=bs, _blk=blk, _r=r, _tv=tv):
                    mv = plsc.load_gather(mean_v.at[_r], [_splat_i32(t)])
                    iv = plsc.load_gather(inv_v.at[_r], [_splat_i32(t)])
                    for v in range(VB):
                        sl = pl.ds((_blk * VB + v) * LANES, LANES)
                        x = _tv[t, sl]
                        _tv[t, sl] = (x - mv) * iv * _gs[v] + _bs[v]
                    return carry

                lax.fori_loop(0, K, pass2, 0)

    fire_chunk(0, 0)

    def outer(cc, carry):
        for p in (0, 1):
            c = cc * 2 + p
            @pl.when(c + 1 < NCHUNK)
            def _prefetch():
                @pl.when(c >= 1)
                def _drain_out():
                    wait_outs(1 - p)

                fire_chunk(c + 1, 1 - p)

            wait_gathers(p)
            compute_chunk(p)
            pb = c * K
            pltpu.async_copy(tokA_v.at[p], out_hbm.at[pl.ds(baseA + pb, K)], sem_outA[p])
            pltpu.async_copy(tokB_v.at[p], out_hbm.at[pl.ds(baseB + pb, K)], sem_outB[p])
        return carry

    lax.fori_loop(0, NCHUNK // 2, outer, 0)
    wait_outs(0)
    wait_outs(1)


def kernel(input_ids, tok_emb, pos_emb, type_emb, gamma, beta):
    comb, pad_row = _comb_call(pos_emb, type_emb)
    ids = input_ids.reshape(-1).astype(jnp.int32)
    out = _sc_embed(ids, tok_emb, comb, pad_row, gamma, beta)
    return out.reshape(input_ids.shape[0], input_ids.shape[1], D)


# speedup vs baseline: 1.1639x; 1.1639x over previous
import functools

import jax
import jax.numpy as jnp
from jax import lax
from jax.experimental import pallas as pl
from jax.experimental.pallas import tpu as pltpu
from jax.experimental.pallas import tpu_sc as plsc

D = 768
MAXPOS = 1025
B = 64
L = 1024
N = B * L
EPS = 1e-12

NC, NS, LANES = 2, 16, 16
NW = NC * NS
TPW = N // NW
K = 32
NCHUNK = TPW // K
NV = D // LANES
INV_D = 1.0 / D


def _comb_body(pos_ref, type_ref, out_ref):
    row = lax.broadcasted_iota(jnp.int32, (MAXPOS, D), 0)
    t0 = type_ref[0:1, :]
    t1 = type_ref[1:2, :]
    out_ref[...] = pos_ref[...] + jnp.where(row == 0, t0, t1)


_comb_call = pl.pallas_call(
    _comb_body,
    out_shape=jax.ShapeDtypeStruct((MAXPOS, D), jnp.float32),
)


def _rsqrt16(a):
    yi = plsc.bitcast(a, jnp.int32)
    magic = jnp.full((LANES,), 0x5F3759DF, dtype=jnp.int32)
    y = plsc.bitcast(magic - lax.shift_right_logical(yi, 1), jnp.float32)
    half = a * 0.5
    for _ in range(3):
        y = y * (1.5 - half * y * y)
    return y


def _splat_i32(x):
    return jnp.full((LANES,), x, dtype=jnp.int32)


_mesh = plsc.VectorSubcoreMesh(core_axis_name="c", subcore_axis_name="s")


@functools.partial(
    pl.kernel,
    mesh=_mesh,
    compiler_params=pltpu.CompilerParams(needs_layout_passes=False),
    out_type=jax.ShapeDtypeStruct((N, D), jnp.float32),
    scratch_types=[
        pltpu.VMEM((TPW,), jnp.int32),
        pltpu.VMEM((TPW,), jnp.int32),
        pltpu.VMEM((2, K, D), jnp.float32),
        pltpu.VMEM((2, K, D), jnp.float32),
        pltpu.VMEM((K, LANES), jnp.float32),
        pltpu.VMEM((K, LANES), jnp.float32),
        pltpu.VMEM((K,), jnp.float32),
        pltpu.VMEM((K,), jnp.float32),
        pltpu.VMEM((D,), jnp.float32),
        pltpu.VMEM((D,), jnp.float32),
        pltpu.SemaphoreType.DMA,
        pltpu.SemaphoreType.DMA,
        pltpu.SemaphoreType.DMA,
        pltpu.SemaphoreType.DMA,
        pltpu.SemaphoreType.DMA,
        pltpu.SemaphoreType.DMA,
    ],
)
def _sc_embed(ids_hbm, tok_hbm, comb_hbm, gamma_hbm, beta_hbm, out_hbm,
              ids_v, pidx_v, tok_v, comb_v, sb_v, ssb_v, mean_v, inv_v,
              gam_v, bet_v, st0, st1, sc0, sc1, so0, so1):
    wid = lax.axis_index("s") * NC + lax.axis_index("c")
    base = wid * TPW
    sem_tok = (st0, st1)
    sem_comb = (sc0, sc1)
    sem_out = (so0, so1)
    pltpu.sync_copy(gamma_hbm, gam_v)
    pltpu.sync_copy(beta_hbm, bet_v)

    pltpu.sync_copy(ids_hbm.at[pl.ds(base, TPW)], ids_v)
    for g in range(TPW // LANES):
        idv = ids_v[pl.ds(g * LANES, LANES)]
        pos = (g * LANES) % L + 1 + lax.iota(jnp.int32, LANES)
        pidx_v[pl.ds(g * LANES, LANES)] = jnp.where(idv == 0, 0, pos)

    zero = jnp.zeros((LANES,), jnp.float32)

    def fire_gathers(c, p):
        off = c * K
        pltpu.async_copy(tok_hbm.at[ids_v.at[pl.ds(off, K)]], tok_v.at[p],
                         sem_tok[p])
        pltpu.async_copy(comb_hbm.at[pidx_v.at[pl.ds(off, K)]], comb_v.at[p],
                         sem_comb[p])

    def wait_gathers(p):
        pltpu.make_async_copy(tok_hbm.at[pl.ds(0, K)], tok_v.at[p], sem_tok[p]).wait()
        pltpu.make_async_copy(tok_hbm.at[pl.ds(0, K)], comb_v.at[p], sem_comb[p]).wait()

    def wait_out(p):
        pltpu.make_async_copy(tok_hbm.at[pl.ds(0, K)], tok_v.at[p], sem_out[p]).wait()

    def compute_chunk(p):
        tv = tok_v.at[p]
        cv = comb_v.at[p]

        def pass1(t, carry):
            s = zero
            ss = zero
            for v in range(NV):
                sl = pl.ds(v * LANES, LANES)
                x = tv[t, sl] + cv[t, sl]
                tv[t, sl] = x
                s = s + x
                ss = ss + x * x
            sb_v[t, :] = s
            ssb_v[t, :] = ss
            return carry

        lax.fori_loop(0, K, pass1, 0)

        for g in range(K // LANES):
            rows = g * LANES + lax.iota(jnp.int32, LANES)
            s_tot = zero
            ss_tot = zero
            for j in range(LANES):
                col = _splat_i32(j)
                s_tot = s_tot + plsc.load_gather(sb_v, [rows, col])
                ss_tot = ss_tot + plsc.load_gather(ssb_v, [rows, col])
            mean = s_tot * INV_D
            var = ss_tot * INV_D - mean * mean
            mean_v[pl.ds(g * LANES, LANES)] = mean
            inv_v[pl.ds(g * LANES, LANES)] = _rsqrt16(var + EPS)

        NBLK = 4
        VB = NV // NBLK
        for blk in range(NBLK):
            gs = [gam_v[pl.ds((blk * VB + v) * LANES, LANES)] for v in range(VB)]
            bs = [bet_v[pl.ds((blk * VB + v) * LANES, LANES)] for v in range(VB)]

            def pass2(t, carry, _gs=gs, _bs=bs, _blk=blk):
                mv = plsc.load_gather(mean_v, [_splat_i32(t)])
                iv = plsc.load_gather(inv_v, [_splat_i32(t)])
                for v in range(VB):
                    sl = pl.ds((_blk * VB + v) * LANES, LANES)
                    x = tv[t, sl]
                    tv[t, sl] = (x - mv) * iv * _gs[v] + _bs[v]
                return carry

            lax.fori_loop(0, K, pass2, 0)

    fire_gathers(0, 0)

    def outer(cc, carry):
        for p in (0, 1):
            c = cc * 2 + p
            @pl.when(c + 1 < NCHUNK)
            def _prefetch():
                @pl.when(c >= 1)
                def _drain_out():
                    wait_out(1 - p)

                fire_gathers(c + 1, 1 - p)

            wait_gathers(p)
            compute_chunk(p)
            cbase = base + c * K
            pltpu.async_copy(tok_v.at[p], out_hbm.at[pl.ds(cbase, K)], sem_out[p])
        return carry

    lax.fori_loop(0, NCHUNK // 2, outer, 0)
    wait_out(0)
    wait_out(1)


def kernel(input_ids, tok_emb, pos_emb, type_emb, gamma, beta):
    comb = _comb_call(pos_emb, type_emb)
    ids = input_ids.reshape(-1).astype(jnp.int32)
    out = _sc_embed(ids, tok_emb, comb, gamma, beta)
    return out.reshape(input_ids.shape[0], input_ids.shape[1], D)
